# baseline (speedup 1.0000x reference)
import functools
import math

import jax
import jax.numpy as jnp
from jax import lax
from jax.experimental import pallas as pl
from jax.experimental.pallas import tpu as pltpu
from jax.experimental.pallas import tpu_sc as plsc

D_MODEL = 64
SCALE = math.sqrt(D_MODEL)


def _build(B, H):
    NW = 32
    assert B == 128 * NW and H % 2 == 0
    mesh = plsc.VectorSubcoreMesh(core_axis_name="c", subcore_axis_name="s")

    @functools.partial(
        pl.kernel,
        mesh=mesh,
        out_type=jax.ShapeDtypeStruct((H, D_MODEL, B), jnp.float32),
        compiler_params=pltpu.CompilerParams(
            use_tc_tiling_on_sc=True, needs_layout_passes=False),
        scratch_types=[
            pltpu.VMEM((H, 128), jnp.int32),
            pltpu.VMEM((H, 128), jnp.int32),
            pltpu.VMEM((2, 128, 128), jnp.float32),
            pltpu.VMEM((2, D_MODEL, 128), jnp.float32),
            pltpu.SemaphoreType.DMA,
            pltpu.SemaphoreType.DMA,
            pltpu.SemaphoreType.DMA,
            pltpu.SemaphoreType.DMA,
        ],
    )
    def emb(xT_hbm, tab_hbm, out_hbm, pidx_v, colb_v, rows_v, outb_v,
            sem0, sem1, osem0, osem1):
        cid = lax.axis_index("c")
        sid = lax.axis_index("s")
        wid = sid * 2 + cid
        j0 = wid * 128
        iota = lax.iota(jnp.int32, 16)
        d16 = jnp.broadcast_to(jnp.int32(0), (16,))

        pltpu.sync_copy(xT_hbm.at[pl.ds(0, H), pl.ds(j0, 128)], pidx_v)

        def prep_r(r, c):
            def prep_q(q, c2):
                v = pidx_v[r, pl.ds(16 * q, 16)]
                colb_v[r, pl.ds(16 * q, 16)] = (v & 1) * 64
                pidx_v[r, pl.ds(16 * q, 16)] = v >> 1
                return c2
            return lax.fori_loop(0, 8, prep_q, c)

        lax.fori_loop(0, H, prep_r, 0)

        def gather(h, buf, sem):
            return pltpu.async_copy(
                tab_hbm.at[pidx_v.at[h]], rows_v.at[buf], sem)

        def gather_wait(buf, sem):
            pltpu.make_async_copy(
                tab_hbm.at[pidx_v.at[0]], rows_v.at[buf], sem).wait()

        def compute(h, buf, osem, first):
            @pl.when(jnp.logical_not(first))
            def _():
                pltpu.make_async_copy(
                    out_hbm.at[0, pl.ds(0, D_MODEL), pl.ds(j0, 128)],
                    outb_v.at[buf], osem).wait()
            rows = rows_v.at[buf]
            outb = outb_v.at[buf]

            def bq_body(bq, c):
                brow = 16 * bq + iota
                colb = colb_v[h, pl.ds(16 * bq, 16)]

                def d_body(d, cb):
                    val = plsc.load_gather(rows, [brow, cb + d])
                    plsc.store_scatter(outb, [d16 + d, brow], val * SCALE)
                    return cb

                lax.fori_loop(0, D_MODEL, d_body, colb, unroll=4)
                return c

            lax.fori_loop(0, 8, bq_body, 0)
            pltpu.async_copy(
                outb, out_hbm.at[h, pl.ds(0, D_MODEL), pl.ds(j0, 128)], osem)

        gather(0, 0, sem0)

        def pair(t, c):
            h0 = 2 * t
            gather(h0 + 1, 1, sem1)
            gather_wait(0, sem0)
            compute(h0, 0, osem0, t == 0)

            @pl.when(t < H // 2 - 1)
            def _():
                gather(h0 + 2, 0, sem0)
            gather_wait(1, sem1)
            compute(h0 + 1, 1, osem1, t == 0)
            return c

        lax.fori_loop(0, H // 2, pair, 0)
        for buf, osem in ((0, osem0), (1, osem1)):
            pltpu.make_async_copy(
                out_hbm.at[0, pl.ds(0, D_MODEL), pl.ds(j0, 128)],
                outb_v.at[buf], osem).wait()

    return emb


def kernel(x, table):
    B, H = x.shape
    V = table.shape[0]
    tab2 = table.reshape(V // 2, 128)
    out = _build(B, H)(x.T, tab2)
    return out.transpose(2, 0, 1)


# speedup vs baseline: 1.4253x; 1.4253x over previous
import functools
import math

import jax
import jax.numpy as jnp
from jax import lax
from jax.experimental import pallas as pl
from jax.experimental.pallas import tpu as pltpu
from jax.experimental.pallas import tpu_sc as plsc

D_MODEL = 64
SCALE = math.sqrt(D_MODEL)


def _build(B, H):
    NW = 32
    assert B == 128 * NW and H % 2 == 0
    mesh = plsc.VectorSubcoreMesh(core_axis_name="c", subcore_axis_name="s")

    @functools.partial(
        pl.kernel,
        mesh=mesh,
        out_type=jax.ShapeDtypeStruct((H, D_MODEL, B), jnp.float32),
        compiler_params=pltpu.CompilerParams(
            use_tc_tiling_on_sc=True, needs_layout_passes=False),
        scratch_types=[
            pltpu.VMEM((H, 128), jnp.int32),
            pltpu.VMEM((H, 128), jnp.int32),
            pltpu.VMEM((2, 128, 128), jnp.float32),
            pltpu.VMEM((2, D_MODEL, 128), jnp.float32),
            pltpu.SemaphoreType.DMA,
            pltpu.SemaphoreType.DMA,
            pltpu.SemaphoreType.DMA,
            pltpu.SemaphoreType.DMA,
        ],
    )
    def emb(xT_hbm, tab_hbm, out_hbm, pidx_v, colb_v, rows_v, outb_v,
            sem0, sem1, osem0, osem1):
        cid = lax.axis_index("c")
        sid = lax.axis_index("s")
        wid = sid * 2 + cid
        j0 = wid * 128
        iota = lax.iota(jnp.int32, 16)
        d16 = jnp.broadcast_to(jnp.int32(0), (16,))

        pltpu.sync_copy(xT_hbm.at[pl.ds(0, H), pl.ds(j0, 128)], pidx_v)

        def prep_r(r, c):
            def prep_q(q, c2):
                v = pidx_v[r, pl.ds(16 * q, 16)]
                colb_v[r, pl.ds(16 * q, 16)] = (v & 1) * 64
                pidx_v[r, pl.ds(16 * q, 16)] = v >> 1
                return c2
            return lax.fori_loop(0, 8, prep_q, c)

        lax.fori_loop(0, H, prep_r, 0)

        def gather(h, buf, sem):
            return pltpu.async_copy(
                tab_hbm.at[pidx_v.at[h]], rows_v.at[buf], sem)

        def gather_wait(buf, sem):
            pltpu.make_async_copy(
                tab_hbm.at[pidx_v.at[0]], rows_v.at[buf], sem).wait()

        def compute(h, buf, osem, first):
            @pl.when(jnp.logical_not(first))
            def _():
                pltpu.make_async_copy(
                    out_hbm.at[0, pl.ds(0, D_MODEL), pl.ds(j0, 128)],
                    outb_v.at[buf], osem).wait()
            rows = rows_v.at[buf]
            outb = outb_v.at[buf]

            def bq_body(bq, c):
                brow = 16 * bq + iota
                colb = colb_v[h, pl.ds(16 * bq, 16)]

                @plsc.parallel_loop(0, D_MODEL, unroll=8)
                def d_body(d):
                    val = plsc.load_gather(rows, [brow, colb + d])
                    plsc.store_scatter(outb, [d16 + d, brow], val * SCALE)

                return c

            lax.fori_loop(0, 8, bq_body, 0)
            pltpu.async_copy(
                outb, out_hbm.at[h, pl.ds(0, D_MODEL), pl.ds(j0, 128)], osem)

        gather(0, 0, sem0)

        def pair(t, c):
            h0 = 2 * t
            gather(h0 + 1, 1, sem1)
            gather_wait(0, sem0)
            compute(h0, 0, osem0, t == 0)

            @pl.when(t < H // 2 - 1)
            def _():
                gather(h0 + 2, 0, sem0)
            gather_wait(1, sem1)
            compute(h0 + 1, 1, osem1, t == 0)
            return c

        lax.fori_loop(0, H // 2, pair, 0)
        for buf, osem in ((0, osem0), (1, osem1)):
            pltpu.make_async_copy(
                out_hbm.at[0, pl.ds(0, D_MODEL), pl.ds(j0, 128)],
                outb_v.at[buf], osem).wait()

    return emb


def kernel(x, table):
    B, H = x.shape
    V = table.shape[0]
    tab2 = table.reshape(V // 2, 128)
    out = _build(B, H)(x.T, tab2)
    return out.transpose(2, 0, 1)


# speedup vs baseline: 1.5678x; 1.1000x over previous
import functools
import math

import jax
import jax.numpy as jnp
from jax import lax
from jax.experimental import pallas as pl
from jax.experimental.pallas import tpu as pltpu
from jax.experimental.pallas import tpu_sc as plsc

D_MODEL = 64
SCALE = math.sqrt(D_MODEL)


def _build(B, H):
    NW = 32
    assert B == 128 * NW and H % 2 == 0
    mesh = plsc.VectorSubcoreMesh(core_axis_name="c", subcore_axis_name="s")

    @functools.partial(
        pl.kernel,
        mesh=mesh,
        out_type=jax.ShapeDtypeStruct((H, D_MODEL, B), jnp.float32),
        compiler_params=pltpu.CompilerParams(
            use_tc_tiling_on_sc=True, needs_layout_passes=False),
        scratch_types=[
            pltpu.VMEM((H, 128), jnp.int32),
            pltpu.VMEM((H, 128), jnp.int32),
            pltpu.VMEM((2, 128, 128), jnp.float32),
            pltpu.VMEM((2, D_MODEL, 128), jnp.float32),
            pltpu.SemaphoreType.DMA,
            pltpu.SemaphoreType.DMA,
            pltpu.SemaphoreType.DMA,
            pltpu.SemaphoreType.DMA,
        ],
    )
    def emb(xT_hbm, tab_hbm, out_hbm, pidx_v, colb_v, rows_v, outb_v,
            sem0, sem1, osem0, osem1):
        cid = lax.axis_index("c")
        sid = lax.axis_index("s")
        wid = sid * 2 + cid
        j0 = wid * 128
        iota = lax.iota(jnp.int32, 16)
        d16 = jnp.broadcast_to(jnp.int32(0), (16,))

        pltpu.sync_copy(xT_hbm.at[pl.ds(0, H), pl.ds(j0, 128)], pidx_v)

        def prep_r(r, c):
            def prep_q(q, c2):
                v = pidx_v[r, pl.ds(16 * q, 16)]
                colb_v[r, pl.ds(16 * q, 16)] = (v & 1) * 64
                pidx_v[r, pl.ds(16 * q, 16)] = v >> 1
                return c2
            return lax.fori_loop(0, 8, prep_q, c)

        lax.fori_loop(0, H, prep_r, 0)

        def gather(h, buf, sem):
            return pltpu.async_copy(
                tab_hbm.at[pidx_v.at[h]], rows_v.at[buf], sem)

        def gather_wait(buf, sem):
            pltpu.make_async_copy(
                tab_hbm.at[pidx_v.at[0]], rows_v.at[buf], sem).wait()

        def compute(h, buf, osem, first):
            @pl.when(jnp.logical_not(first))
            def _():
                pltpu.make_async_copy(
                    out_hbm.at[0, pl.ds(0, D_MODEL), pl.ds(j0, 128)],
                    outb_v.at[buf], osem).wait()
            rows = rows_v.at[buf]
            outb = outb_v.at[buf]

            brows = [16 * q + iota for q in range(8)]
            colbs = [colb_v[h, pl.ds(16 * q, 16)] for q in range(8)]

            @plsc.parallel_loop(0, D_MODEL, unroll=4)
            def d_body(d):
                for q in range(8):
                    val = plsc.load_gather(rows, [brows[q], colbs[q] + d])
                    outb[d, pl.ds(16 * q, 16)] = val * SCALE
            pltpu.async_copy(
                outb, out_hbm.at[h, pl.ds(0, D_MODEL), pl.ds(j0, 128)], osem)

        gather(0, 0, sem0)

        def pair(t, c):
            h0 = 2 * t
            gather(h0 + 1, 1, sem1)
            gather_wait(0, sem0)
            compute(h0, 0, osem0, t == 0)

            @pl.when(t < H // 2 - 1)
            def _():
                gather(h0 + 2, 0, sem0)
            gather_wait(1, sem1)
            compute(h0 + 1, 1, osem1, t == 0)
            return c

        lax.fori_loop(0, H // 2, pair, 0)
        for buf, osem in ((0, osem0), (1, osem1)):
            pltpu.make_async_copy(
                out_hbm.at[0, pl.ds(0, D_MODEL), pl.ds(j0, 128)],
                outb_v.at[buf], osem).wait()

    return emb


def kernel(x, table):
    B, H = x.shape
    V = table.shape[0]
    tab2 = table.reshape(V // 2, 128)
    out = _build(B, H)(x.T, tab2)
    return out.transpose(2, 0, 1)
